# Initial kernel scaffold; baseline (speedup 1.0000x reference)
import jax
import jax.numpy as jnp
from jax.experimental import pallas as pl


def kernel(x, token_table, pos_table):
    raise NotImplementedError("write your pallas kernel here")


# speedup vs baseline: 1.4873x; 1.4873x over previous
import functools

import jax
import jax.numpy as jnp
from jax import lax
from jax.experimental import pallas as pl
from jax.experimental.pallas import tpu as pltpu
from jax.experimental.pallas import tpu_sc as plsc

D = 32
SEQ = 200
NW = 32
KSEQ = 4
CH = KSEQ * SEQ
G = 100
NG = CH // G
NBUF = 2

def _mesh():
    return plsc.VectorSubcoreMesh(
        core_axis_name="c", subcore_axis_name="s",
        num_cores=2, num_subcores=16)


def _emb_body(rows_total, x_hbm, tok_hbm, pos_hbm, out_hbm,
              idx_v, rows_v, pos_v, g0, g1, w0, w1):
    rpw = rows_total // NW
    ngrp = rpw // G
    nch = rpw // CH

    cid = lax.axis_index("c")
    sid = lax.axis_index("s")
    wid = sid * 2 + cid
    gbase = wid * ngrp
    rbase = wid * rpw

    pltpu.sync_copy(x_hbm.at[pl.ds(gbase, ngrp)], idx_v)
    pltpu.sync_copy(pos_hbm, pos_v)

    gsems = (g0, g1)
    wsems = (w0, w1)

    def fire(c, b):
        for g in range(NG):
            pltpu.async_copy(
                tok_hbm.at[idx_v.at[c * NG + g]],
                rows_v.at[b, pl.ds(g * G, G)],
                gsems[b])

    def wait_gathers(b):
        for g in range(NG):
            pltpu.make_async_copy(
                tok_hbm.at[idx_v.at[0]],
                rows_v.at[b, pl.ds(g * G, G)],
                gsems[b]).wait()

    def issue_write(c, b):
        pltpu.async_copy(
            rows_v.at[b],
            out_hbm.at[pl.ds(rbase + c * CH, CH)],
            wsems[b])

    def wait_write(b):
        pltpu.make_async_copy(
            rows_v.at[b],
            out_hbm.at[pl.ds(0, CH)],
            wsems[b]).wait()

    def add_pos(b):
        @plsc.parallel_loop(0, SEQ, 1, unroll=2)
        def _(s):
            p0 = pos_v[s, pl.ds(0, 16)]
            p1 = pos_v[s, pl.ds(16, 16)]
            for q in range(KSEQ):
                r = q * SEQ + s
                plsc.addupdate(rows_v.at[b, r, pl.ds(0, 16)], p0)
                plsc.addupdate(rows_v.at[b, r, pl.ds(16, 16)], p1)

    fire(0, 0)

    @pl.loop(0, nch // NBUF)
    def _(ci):
        for b in range(NBUF):
            c = ci * NBUF + b
            nb = 1 - b
            @pl.when(c + 1 < nch)
            def _():
                @pl.when(c >= 1)
                def _():
                    wait_write(nb)
                fire(c + 1, nb)

            wait_gathers(b)
            add_pos(b)
            issue_write(c, b)

    wait_write(0)
    wait_write(1)


def kernel(x, token_table, pos_table):
    batch, seq = x.shape
    rows = batch * seq
    x2d = x.reshape(rows // G, G).astype(jnp.int32)

    kern = functools.partial(
        pl.kernel,
        out_type=jax.ShapeDtypeStruct((rows, D), jnp.float32),
        mesh=_mesh(),
        compiler_params=pltpu.CompilerParams(use_tc_tiling_on_sc=False),
        scratch_types=[
            pltpu.VMEM((rows // NW // G, G), jnp.int32),
            pltpu.VMEM((NBUF, CH, D), jnp.float32),
            pltpu.VMEM((SEQ, D), jnp.float32),
            pltpu.SemaphoreType.DMA,
            pltpu.SemaphoreType.DMA,
            pltpu.SemaphoreType.DMA,
            pltpu.SemaphoreType.DMA,
        ],
    )(functools.partial(_emb_body, rows))

    out = kern(x2d, token_table, pos_table)
    return out.reshape(batch, seq, D)
